# baseline (speedup 1.0000x reference)
import functools

import jax
import jax.numpy as jnp
from jax import lax
from jax.experimental import pallas as pl
from jax.experimental.pallas import tpu as pltpu
from jax.experimental.pallas import tpu_sc as plsc


def _detile(V, D):
    info = plsc.get_sparse_core_info()
    NC, NS, L = info.num_cores, info.num_subcores, info.num_lanes
    NW = NC * NS
    GV = 2 * D
    NG = V // GV
    KMAX = (NG + NW - 1) // NW
    DEPTH = 4
    mesh = plsc.VectorSubcoreMesh(core_axis_name="c", subcore_axis_name="s")

    @functools.partial(
        pl.kernel,
        mesh=mesh,
        compiler_params=pltpu.CompilerParams(use_tc_tiling_on_sc=True),
        out_type=jax.ShapeDtypeStruct((NG * D, GV), jnp.float32),
        scratch_types=[pltpu.SemaphoreType.DMA],
    )
    def k(tT_hbm, out_hbm, sem):
        wid = lax.axis_index("s") * NC + lax.axis_index("c")

        def issue(g):
            pltpu.async_copy(
                tT_hbm.at[pl.ds(0, D), pl.ds(g * GV, GV)],
                out_hbm.at[pl.ds(g * D, D)],
                sem,
            )

        def wait_one():
            pltpu.make_async_copy(
                tT_hbm.at[pl.ds(0, D), pl.ds(0, GV)],
                out_hbm.at[pl.ds(0, D)],
                sem,
            ).wait()

        for p in range(DEPTH):

            @pl.when(wid + p * NW < NG)
            def _():
                issue(wid + p * NW)

        def step(kk, carry):
            @pl.when(wid + kk * NW < NG)
            def _():
                wait_one()

            @pl.when(wid + (kk + DEPTH) * NW < NG)
            def _():
                issue(wid + (kk + DEPTH) * NW)

            return carry

        lax.fori_loop(0, KMAX, step, 0)

    return k


def _transpose_lin(V, D):
    info = plsc.get_sparse_core_info()
    NC, NS, L = info.num_cores, info.num_subcores, info.num_lanes
    NW = NC * NS
    GV = 2 * D
    GD = GV * D
    NG = V // GV
    TAIL_V = V - NG * GV
    K = NG // NW
    EXTRA = NG - K * NW
    assert K % 2 == 0
    mesh = plsc.VectorSubcoreMesh(core_axis_name="c", subcore_axis_name="s")

    @functools.partial(
        pl.kernel,
        mesh=mesh,
        compiler_params=pltpu.CompilerParams(
            use_tc_tiling_on_sc=False, needs_layout_passes=False
        ),
        out_type=jax.ShapeDtypeStruct((V * D,), jnp.float32),
        scratch_types=[
            pltpu.VMEM((2, GD), jnp.float32),
            pltpu.VMEM((2, GD), jnp.float32),
            pltpu.SemaphoreType.DMA,
            pltpu.SemaphoreType.DMA,
            pltpu.SemaphoreType.DMA,
            pltpu.SemaphoreType.DMA,
        ],
    )
    def k(in_hbm, tail_hbm, out_hbm, in_v, out_v, si0, si1, so0, so1):
        wid = lax.axis_index("s") * NC + lax.axis_index("c")
        sin = (si0, si1)
        sout = (so0, so1)
        iota = lax.iota(jnp.int32, L)
        scat = [(c * L + iota) * D for c in range(GV // L)]

        def issue_in(g, buf):
            pltpu.async_copy(
                in_hbm.at[pl.ds(g * GD, GD)], in_v.at[buf], sin[buf]
            )

        def wait_in(buf):
            pltpu.make_async_copy(
                in_hbm.at[pl.ds(0, GD)], in_v.at[buf], sin[buf]
            ).wait()

        def transpose(buf):
            for d in range(D):
                for c in range(GV // L):
                    x = in_v[buf, pl.ds(d * GV + c * L, L)]
                    plsc.store_scatter(out_v.at[buf], [scat[c] + d], x)

        def issue_out(g, buf):
            pltpu.async_copy(
                out_v.at[buf], out_hbm.at[pl.ds(g * GD, GD)], sout[buf]
            )

        def wait_out(buf):
            pltpu.make_async_copy(
                out_v.at[buf], out_hbm.at[pl.ds(0, GD)], sout[buf]
            ).wait()

        issue_in(wid, 0)
        issue_in(wid + NW, 1)

        def step(s, carry):
            for hb in range(2):
                kk = 2 * s + hb
                g = wid + kk * NW
                wait_in(hb)
                transpose(hb)

                @pl.when(s > 0)
                def _():
                    wait_out(hb)

                issue_out(g, hb)

                @pl.when(kk + 2 < K)
                def _():
                    issue_in(g + 2 * NW, hb)

            return carry

        lax.fori_loop(0, K // 2, step, 0)
        wait_out(0)
        wait_out(1)

        if EXTRA > 0:

            @pl.when(wid < EXTRA)
            def _():
                g = K * NW + wid
                issue_in(g, 0)
                wait_in(0)
                transpose(0)
                issue_out(g, 0)
                wait_out(0)

        if TAIL_V > 0:

            @pl.when(wid == NW - 1)
            def _():
                pltpu.sync_copy(
                    tail_hbm, out_hbm.at[pl.ds(NG * GD, TAIL_V * D)]
                )

    return k


def _pooled_lookup(B, T, D):
    info = plsc.get_sparse_core_info()
    NC, NS, L = info.num_cores, info.num_subcores, info.num_lanes
    NW = NC * NS
    assert B % NW == 0 and D % L == 0 and D // L == 4
    BPW = B // NW
    assert BPW % 2 == 0
    G = (T + L - 1) // L
    TAIL = T - (G - 1) * L
    C0 = 104
    C1 = T - C0
    SLAB = BPW * T
    mesh = plsc.VectorSubcoreMesh(core_axis_name="c", subcore_axis_name="s")

    @functools.partial(
        pl.kernel,
        mesh=mesh,
        compiler_params=pltpu.CompilerParams(use_tc_tiling_on_sc=False),
        out_type=jax.ShapeDtypeStruct((B * D,), jnp.float32),
        scratch_types=[
            pltpu.VMEM((SLAB,), jnp.int32),
            pltpu.VMEM((SLAB + L,), jnp.float32),
            pltpu.VMEM((2, T, D), jnp.float32),
            pltpu.VMEM((BPW * D,), jnp.float32),
            pltpu.SemaphoreType.DMA,
            pltpu.SemaphoreType.DMA,
        ],
    )
    def k(tok_hbm, msk_hbm, table_hbm, out_hbm, tok_v, msk_v, rows_v, out_v, sem0, sem1):
        wid = lax.axis_index("s") * NC + lax.axis_index("c")
        slab_base = wid * SLAB
        pltpu.sync_copy(tok_hbm.at[pl.ds(slab_base, SLAB)], tok_v.at[pl.ds(0, SLAB)])
        pltpu.sync_copy(msk_hbm.at[pl.ds(slab_base, SLAB)], msk_v.at[pl.ds(0, SLAB)])
        lane = lax.iota(jnp.int32, L)
        z = jnp.zeros((L,), jnp.float32)
        sems = (sem0, sem1)

        def issue(j, buf_i, sem):
            base = j * T
            pltpu.async_copy(
                table_hbm.at[tok_v.at[pl.ds(base, C0)]],
                rows_v.at[buf_i].at[pl.ds(0, C0)],
                sem,
            )
            pltpu.async_copy(
                table_hbm.at[tok_v.at[pl.ds(base + C0, C1)]],
                rows_v.at[buf_i].at[pl.ds(C0, C1)],
                sem,
            )

        def drain(buf_i, sem):
            pltpu.make_async_copy(
                table_hbm.at[pl.ds(0, C0)], rows_v.at[buf_i].at[pl.ds(0, C0)], sem
            ).wait()
            pltpu.make_async_copy(
                table_hbm.at[pl.ds(0, C1)], rows_v.at[buf_i].at[pl.ds(C0, C1)], sem
            ).wait()

        def reduce_row(j, buf_i):
            buf = rows_v.at[buf_i]
            base = j * T
            a = [z, z, z, z]
            cntv = z
            for g in range(G):
                mvec = msk_v[pl.ds(base + g * L, L)]
                nv = L
                if g == G - 1:
                    mvec = jnp.where(lane < TAIL, mvec, 0.0)
                    nv = TAIL
                cntv = cntv + mvec
                for i in range(nv):
                    t = g * L + i
                    m = mvec[i]
                    for kk in range(4):
                        a[kk] = a[kk] + buf[t, pl.ds(kk * L, L)] * m
            cnt = cntv[0]
            for i in range(1, L):
                cnt = cnt + cntv[i]
            denom = jnp.maximum(z + cnt, 1.0)
            for kk in range(4):
                out_v[pl.ds(j * D + kk * L, L)] = a[kk] / denom

        issue(0, 0, sem0)
        issue(1, 1, sem1)

        def step(s, carry):
            for half in range(2):
                j = 2 * s + half
                drain(half, sems[half])
                reduce_row(j, half)

                @pl.when(s < BPW // 2 - 1)
                def _():
                    issue(j + 2, half, sems[half])

            return carry

        lax.fori_loop(0, BPW // 2, step, 0)
        pltpu.sync_copy(out_v, out_hbm.at[pl.ds(wid * BPW * D, BPW * D)])

    return k


def kernel(token_ids, mask, table):
    B, T = token_ids.shape
    V, D = table.shape
    tok_flat = token_ids.astype(jnp.int32).reshape(-1)
    mask_flat = mask.astype(jnp.float32).reshape(-1)
    GV = 2 * D
    tail_v = V % GV
    tail = table[V - tail_v :].reshape(-1)
    inter = _detile(V, D)(table.T).reshape(-1)
    table_lin = _transpose_lin(V, D)(inter, tail).reshape(V, D)
    out_flat = _pooled_lookup(B, T, D)(tok_flat, mask_flat, table_lin)
    return out_flat.reshape(B, D)


# speedup vs baseline: 6.0813x; 6.0813x over previous
import functools

import jax
import jax.numpy as jnp
from jax import lax
from jax.experimental import pallas as pl
from jax.experimental.pallas import tpu as pltpu
from jax.experimental.pallas import tpu_sc as plsc


def _tc_transpose(V, D):
    BC = 512
    NB = (V + BC - 1) // BC

    def body(in_ref, o_ref, scr):
        scr[...] = in_ref[...].T
        o_ref[:, 0:D] = scr[pl.Slice(0, BC // 2, 2), :]
        o_ref[:, D : 2 * D] = scr[pl.Slice(1, BC // 2, 2), :]

    return pl.pallas_call(
        body,
        grid=(NB,),
        in_specs=[pl.BlockSpec((D, BC), lambda i: (0, i))],
        out_specs=pl.BlockSpec((BC // 2, 2 * D), lambda i: (i, 0)),
        out_shape=jax.ShapeDtypeStruct((V // 2, 2 * D), jnp.float32),
        scratch_shapes=[pltpu.VMEM((BC, D), jnp.float32)],
    )


def _pooled_lookup(B, T, D):
    info = plsc.get_sparse_core_info()
    NC, NS, L = info.num_cores, info.num_subcores, info.num_lanes
    NW = NC * NS
    assert B % NW == 0 and D % L == 0 and D // L == 4
    BPW = B // NW
    assert BPW % 2 == 0
    G = (T + L - 1) // L
    TAIL = T - (G - 1) * L
    C0 = 104
    C1 = T - C0
    SLAB = BPW * T
    mesh = plsc.VectorSubcoreMesh(core_axis_name="c", subcore_axis_name="s")

    @functools.partial(
        pl.kernel,
        mesh=mesh,
        compiler_params=pltpu.CompilerParams(use_tc_tiling_on_sc=False),
        out_type=jax.ShapeDtypeStruct((B * D,), jnp.float32),
        scratch_types=[
            pltpu.VMEM((SLAB,), jnp.int32),
            pltpu.VMEM((SLAB + L,), jnp.float32),
            pltpu.VMEM((2, T, D), jnp.float32),
            pltpu.VMEM((BPW * D,), jnp.float32),
            pltpu.SemaphoreType.DMA,
            pltpu.SemaphoreType.DMA,
        ],
    )
    def k(tok_hbm, msk_hbm, table_hbm, out_hbm, tok_v, msk_v, rows_v, out_v, sem0, sem1):
        wid = lax.axis_index("s") * NC + lax.axis_index("c")
        slab_base = wid * SLAB
        pltpu.sync_copy(tok_hbm.at[pl.ds(slab_base, SLAB)], tok_v.at[pl.ds(0, SLAB)])
        pltpu.sync_copy(msk_hbm.at[pl.ds(slab_base, SLAB)], msk_v.at[pl.ds(0, SLAB)])
        lane = lax.iota(jnp.int32, L)
        z = jnp.zeros((L,), jnp.float32)
        sems = (sem0, sem1)

        def issue(j, buf_i, sem):
            base = j * T
            pltpu.async_copy(
                table_hbm.at[tok_v.at[pl.ds(base, C0)]],
                rows_v.at[buf_i].at[pl.ds(0, C0)],
                sem,
            )
            pltpu.async_copy(
                table_hbm.at[tok_v.at[pl.ds(base + C0, C1)]],
                rows_v.at[buf_i].at[pl.ds(C0, C1)],
                sem,
            )

        def drain(buf_i, sem):
            pltpu.make_async_copy(
                table_hbm.at[pl.ds(0, C0)], rows_v.at[buf_i].at[pl.ds(0, C0)], sem
            ).wait()
            pltpu.make_async_copy(
                table_hbm.at[pl.ds(0, C1)], rows_v.at[buf_i].at[pl.ds(C0, C1)], sem
            ).wait()

        def reduce_row(j, buf_i):
            buf = rows_v.at[buf_i]
            base = j * T
            a = [z, z, z, z]
            cntv = z
            for g in range(G):
                mvec = msk_v[pl.ds(base + g * L, L)]
                nv = L
                if g == G - 1:
                    mvec = jnp.where(lane < TAIL, mvec, 0.0)
                    nv = TAIL
                cntv = cntv + mvec
                for i in range(nv):
                    t = g * L + i
                    m = mvec[i]
                    for kk in range(4):
                        a[kk] = a[kk] + buf[t, pl.ds(kk * L, L)] * m
            cnt = cntv[0]
            for i in range(1, L):
                cnt = cnt + cntv[i]
            denom = jnp.maximum(z + cnt, 1.0)
            for kk in range(4):
                out_v[pl.ds(j * D + kk * L, L)] = a[kk] / denom

        issue(0, 0, sem0)
        issue(1, 1, sem1)

        def step(s, carry):
            for half in range(2):
                j = 2 * s + half
                drain(half, sems[half])
                reduce_row(j, half)

                @pl.when(s < BPW // 2 - 1)
                def _():
                    issue(j + 2, half, sems[half])

            return carry

        lax.fori_loop(0, BPW // 2, step, 0)
        pltpu.sync_copy(out_v, out_hbm.at[pl.ds(wid * BPW * D, BPW * D)])

    return k


def kernel(token_ids, mask, table):
    B, T = token_ids.shape
    V, D = table.shape
    tok_flat = token_ids.astype(jnp.int32).reshape(-1)
    mask_flat = mask.astype(jnp.float32).reshape(-1)
    table_lin = _tc_transpose(V, D)(table.T).reshape(V, D)
    out_flat = _pooled_lookup(B, T, D)(tok_flat, mask_flat, table_lin)
    return out_flat.reshape(B, D)


# speedup vs baseline: 11.9570x; 1.9662x over previous
import functools

import jax
import jax.numpy as jnp
from jax import lax
from jax.experimental import pallas as pl
from jax.experimental.pallas import tpu as pltpu
from jax.experimental.pallas import tpu_sc as plsc


def _tc_transpose(V, D):
    BC = 2048
    NB = (V + BC - 1) // BC

    def body(in_ref, o_ref, scr):
        scr[...] = in_ref[...].T
        o_ref[:, 0:D] = scr[pl.Slice(0, BC // 2, 2), :]
        o_ref[:, D : 2 * D] = scr[pl.Slice(1, BC // 2, 2), :]

    return pl.pallas_call(
        body,
        grid=(NB,),
        in_specs=[pl.BlockSpec((D, BC), lambda i: (0, i))],
        out_specs=pl.BlockSpec((BC // 2, 2 * D), lambda i: (i, 0)),
        out_shape=jax.ShapeDtypeStruct((V // 2, 2 * D), jnp.float32),
        scratch_shapes=[pltpu.VMEM((BC, D), jnp.float32)],
    )


def _pooled_lookup(B, T, D):
    info = plsc.get_sparse_core_info()
    NC, NS, L = info.num_cores, info.num_subcores, info.num_lanes
    NW = NC * NS
    assert B % NW == 0 and D % L == 0 and D // L == 4
    BPW = B // NW
    assert BPW % 2 == 0
    G = (T + L - 1) // L
    TAIL = T - (G - 1) * L
    C0 = 104
    C1 = T - C0
    SLAB = BPW * T
    mesh = plsc.VectorSubcoreMesh(core_axis_name="c", subcore_axis_name="s")

    @functools.partial(
        pl.kernel,
        mesh=mesh,
        compiler_params=pltpu.CompilerParams(use_tc_tiling_on_sc=False),
        out_type=jax.ShapeDtypeStruct((B * D,), jnp.float32),
        scratch_types=[
            pltpu.VMEM((SLAB,), jnp.int32),
            pltpu.VMEM((SLAB + L,), jnp.float32),
            pltpu.VMEM((2, T, D), jnp.float32),
            pltpu.VMEM((BPW * D,), jnp.float32),
            pltpu.SemaphoreType.DMA,
            pltpu.SemaphoreType.DMA,
        ],
    )
    def k(tok_hbm, msk_hbm, table_hbm, out_hbm, tok_v, msk_v, rows_v, out_v, sem0, sem1):
        wid = lax.axis_index("s") * NC + lax.axis_index("c")
        slab_base = wid * SLAB
        pltpu.sync_copy(tok_hbm.at[pl.ds(slab_base, SLAB)], tok_v.at[pl.ds(0, SLAB)])
        pltpu.sync_copy(msk_hbm.at[pl.ds(slab_base, SLAB)], msk_v.at[pl.ds(0, SLAB)])
        lane = lax.iota(jnp.int32, L)
        z = jnp.zeros((L,), jnp.float32)
        sems = (sem0, sem1)

        def issue(j, buf_i, sem):
            base = j * T
            pltpu.async_copy(
                table_hbm.at[tok_v.at[pl.ds(base, C0)]],
                rows_v.at[buf_i].at[pl.ds(0, C0)],
                sem,
            )
            pltpu.async_copy(
                table_hbm.at[tok_v.at[pl.ds(base + C0, C1)]],
                rows_v.at[buf_i].at[pl.ds(C0, C1)],
                sem,
            )

        def drain(buf_i, sem):
            pltpu.make_async_copy(
                table_hbm.at[pl.ds(0, C0)], rows_v.at[buf_i].at[pl.ds(0, C0)], sem
            ).wait()
            pltpu.make_async_copy(
                table_hbm.at[pl.ds(0, C1)], rows_v.at[buf_i].at[pl.ds(C0, C1)], sem
            ).wait()

        def reduce_row(j, buf_i):
            buf = rows_v.at[buf_i]
            base = j * T
            a = [z, z, z, z]
            cntv = z
            for g in range(G):
                mvec = msk_v[pl.ds(base + g * L, L)]
                nv = L
                if g == G - 1:
                    mvec = jnp.where(lane < TAIL, mvec, 0.0)
                    nv = TAIL
                cntv = cntv + mvec
                for i in range(nv):
                    t = g * L + i
                    m = mvec[i]
                    for kk in range(4):
                        a[kk] = a[kk] + buf[t, pl.ds(kk * L, L)] * m
            cnt = cntv[0]
            for i in range(1, L):
                cnt = cnt + cntv[i]
            denom = jnp.maximum(z + cnt, 1.0)
            for kk in range(4):
                out_v[pl.ds(j * D + kk * L, L)] = a[kk] / denom

        issue(0, 0, sem0)
        issue(1, 1, sem1)

        def step(s, carry):
            for half in range(2):
                j = 2 * s + half
                drain(half, sems[half])
                reduce_row(j, half)

                @pl.when(s < BPW // 2 - 1)
                def _():
                    issue(j + 2, half, sems[half])

            return carry

        lax.fori_loop(0, BPW // 2, step, 0)
        pltpu.sync_copy(out_v, out_hbm.at[pl.ds(wid * BPW * D, BPW * D)])

    return k


def kernel(token_ids, mask, table):
    B, T = token_ids.shape
    V, D = table.shape
    tok_flat = token_ids.astype(jnp.int32).reshape(-1)
    mask_flat = mask.astype(jnp.float32).reshape(-1)
    table_lin = _tc_transpose(V, D)(table.T).reshape(V, D)
    out_flat = _pooled_lookup(B, T, D)(tok_flat, mask_flat, table_lin)
    return out_flat.reshape(B, D)


# speedup vs baseline: 15.8505x; 1.3256x over previous
import functools

import jax
import jax.numpy as jnp
from jax import lax
from jax.experimental import pallas as pl
from jax.experimental.pallas import tpu as pltpu
from jax.experimental.pallas import tpu_sc as plsc


def _tc_transpose(V, D):
    BC = 8192
    NB = (V + BC - 1) // BC

    def body(in_ref, o_ref, scr):
        scr[...] = in_ref[...].T
        o_ref[:, 0:D] = scr[pl.Slice(0, BC // 2, 2), :]
        o_ref[:, D : 2 * D] = scr[pl.Slice(1, BC // 2, 2), :]

    return pl.pallas_call(
        body,
        grid=(NB,),
        in_specs=[pl.BlockSpec((D, BC), lambda i: (0, i))],
        out_specs=pl.BlockSpec((BC // 2, 2 * D), lambda i: (i, 0)),
        out_shape=jax.ShapeDtypeStruct((V // 2, 2 * D), jnp.float32),
        scratch_shapes=[pltpu.VMEM((BC, D), jnp.float32)],
    )


def _pooled_lookup(B, T, D):
    info = plsc.get_sparse_core_info()
    NC, NS, L = info.num_cores, info.num_subcores, info.num_lanes
    NW = NC * NS
    assert B % NW == 0 and D % L == 0 and D // L == 4
    BPW = B // NW
    assert BPW % 2 == 0
    G = (T + L - 1) // L
    TAIL = T - (G - 1) * L
    C0 = 104
    C1 = T - C0
    SLAB = BPW * T
    mesh = plsc.VectorSubcoreMesh(core_axis_name="c", subcore_axis_name="s")

    @functools.partial(
        pl.kernel,
        mesh=mesh,
        compiler_params=pltpu.CompilerParams(use_tc_tiling_on_sc=False),
        out_type=jax.ShapeDtypeStruct((B * D,), jnp.float32),
        scratch_types=[
            pltpu.VMEM((SLAB,), jnp.int32),
            pltpu.VMEM((SLAB + L,), jnp.float32),
            pltpu.VMEM((2, T, D), jnp.float32),
            pltpu.VMEM((BPW * D,), jnp.float32),
            pltpu.SemaphoreType.DMA,
            pltpu.SemaphoreType.DMA,
        ],
    )
    def k(tok_hbm, msk_hbm, table_hbm, out_hbm, tok_v, msk_v, rows_v, out_v, sem0, sem1):
        wid = lax.axis_index("s") * NC + lax.axis_index("c")
        slab_base = wid * SLAB
        pltpu.sync_copy(tok_hbm.at[pl.ds(slab_base, SLAB)], tok_v.at[pl.ds(0, SLAB)])
        pltpu.sync_copy(msk_hbm.at[pl.ds(slab_base, SLAB)], msk_v.at[pl.ds(0, SLAB)])
        lane = lax.iota(jnp.int32, L)
        z = jnp.zeros((L,), jnp.float32)
        sems = (sem0, sem1)

        def issue(j, buf_i, sem):
            base = j * T
            pltpu.async_copy(
                table_hbm.at[tok_v.at[pl.ds(base, C0)]],
                rows_v.at[buf_i].at[pl.ds(0, C0)],
                sem,
            )
            pltpu.async_copy(
                table_hbm.at[tok_v.at[pl.ds(base + C0, C1)]],
                rows_v.at[buf_i].at[pl.ds(C0, C1)],
                sem,
            )

        def drain(buf_i, sem):
            pltpu.make_async_copy(
                table_hbm.at[pl.ds(0, C0)], rows_v.at[buf_i].at[pl.ds(0, C0)], sem
            ).wait()
            pltpu.make_async_copy(
                table_hbm.at[pl.ds(0, C1)], rows_v.at[buf_i].at[pl.ds(C0, C1)], sem
            ).wait()

        def reduce_row(j, buf_i):
            buf = rows_v.at[buf_i]
            base = j * T
            a = [z, z, z, z]
            cntv = z
            for g in range(G):
                mvec = msk_v[pl.ds(base + g * L, L)]
                nv = L
                if g == G - 1:
                    mvec = jnp.where(lane < TAIL, mvec, 0.0)
                    nv = TAIL
                cntv = cntv + mvec
                for i in range(nv):
                    t = g * L + i
                    m = mvec[i]
                    for kk in range(4):
                        a[kk] = a[kk] + buf[t, pl.ds(kk * L, L)] * m
            cnt = cntv[0]
            for i in range(1, L):
                cnt = cnt + cntv[i]
            denom = jnp.maximum(z + cnt, 1.0)
            for kk in range(4):
                out_v[pl.ds(j * D + kk * L, L)] = a[kk] / denom

        issue(0, 0, sem0)
        issue(1, 1, sem1)

        def step(s, carry):
            for half in range(2):
                j = 2 * s + half
                drain(half, sems[half])
                reduce_row(j, half)

                @pl.when(s < BPW // 2 - 1)
                def _():
                    issue(j + 2, half, sems[half])

            return carry

        lax.fori_loop(0, BPW // 2, step, 0)
        pltpu.sync_copy(out_v, out_hbm.at[pl.ds(wid * BPW * D, BPW * D)])

    return k


def kernel(token_ids, mask, table):
    B, T = token_ids.shape
    V, D = table.shape
    tok_flat = token_ids.astype(jnp.int32).reshape(-1)
    mask_flat = mask.astype(jnp.float32).reshape(-1)
    table_lin = _tc_transpose(V, D)(table.T).reshape(V, D)
    out_flat = _pooled_lookup(B, T, D)(tok_flat, mask_flat, table_lin)
    return out_flat.reshape(B, D)


# speedup vs baseline: 16.2783x; 1.0270x over previous
import functools

import jax
import jax.numpy as jnp
from jax import lax
from jax.experimental import pallas as pl
from jax.experimental.pallas import tpu as pltpu
from jax.experimental.pallas import tpu_sc as plsc


def _tc_transpose(V, D):
    BC = 16384
    NB = (V + BC - 1) // BC

    def body(in_ref, o_ref, scr):
        scr[...] = in_ref[...].T
        o_ref[:, 0:D] = scr[pl.Slice(0, BC // 2, 2), :]
        o_ref[:, D : 2 * D] = scr[pl.Slice(1, BC // 2, 2), :]

    return pl.pallas_call(
        body,
        grid=(NB,),
        in_specs=[pl.BlockSpec((D, BC), lambda i: (0, i))],
        out_specs=pl.BlockSpec((BC // 2, 2 * D), lambda i: (i, 0)),
        out_shape=jax.ShapeDtypeStruct((V // 2, 2 * D), jnp.float32),
        scratch_shapes=[pltpu.VMEM((BC, D), jnp.float32)],
    )


def _pooled_lookup(B, T, D):
    info = plsc.get_sparse_core_info()
    NC, NS, L = info.num_cores, info.num_subcores, info.num_lanes
    NW = NC * NS
    assert B % NW == 0 and D % L == 0 and D // L == 4
    BPW = B // NW
    assert BPW % 2 == 0
    G = (T + L - 1) // L
    TAIL = T - (G - 1) * L
    C0 = 104
    C1 = T - C0
    SLAB = BPW * T
    mesh = plsc.VectorSubcoreMesh(core_axis_name="c", subcore_axis_name="s")

    @functools.partial(
        pl.kernel,
        mesh=mesh,
        compiler_params=pltpu.CompilerParams(use_tc_tiling_on_sc=False),
        out_type=jax.ShapeDtypeStruct((B * D,), jnp.float32),
        scratch_types=[
            pltpu.VMEM((SLAB,), jnp.int32),
            pltpu.VMEM((SLAB + L,), jnp.float32),
            pltpu.VMEM((2, T, D), jnp.float32),
            pltpu.VMEM((BPW * D,), jnp.float32),
            pltpu.SemaphoreType.DMA,
            pltpu.SemaphoreType.DMA,
        ],
    )
    def k(tok_hbm, msk_hbm, table_hbm, out_hbm, tok_v, msk_v, rows_v, out_v, sem0, sem1):
        wid = lax.axis_index("s") * NC + lax.axis_index("c")
        slab_base = wid * SLAB
        pltpu.sync_copy(tok_hbm.at[pl.ds(slab_base, SLAB)], tok_v.at[pl.ds(0, SLAB)])
        pltpu.sync_copy(msk_hbm.at[pl.ds(slab_base, SLAB)], msk_v.at[pl.ds(0, SLAB)])
        lane = lax.iota(jnp.int32, L)
        z = jnp.zeros((L,), jnp.float32)
        sems = (sem0, sem1)

        def issue(j, buf_i, sem):
            base = j * T
            pltpu.async_copy(
                table_hbm.at[tok_v.at[pl.ds(base, C0)]],
                rows_v.at[buf_i].at[pl.ds(0, C0)],
                sem,
            )
            pltpu.async_copy(
                table_hbm.at[tok_v.at[pl.ds(base + C0, C1)]],
                rows_v.at[buf_i].at[pl.ds(C0, C1)],
                sem,
            )

        def drain(buf_i, sem):
            pltpu.make_async_copy(
                table_hbm.at[pl.ds(0, C0)], rows_v.at[buf_i].at[pl.ds(0, C0)], sem
            ).wait()
            pltpu.make_async_copy(
                table_hbm.at[pl.ds(0, C1)], rows_v.at[buf_i].at[pl.ds(C0, C1)], sem
            ).wait()

        def reduce_row(j, buf_i):
            buf = rows_v.at[buf_i]
            base = j * T
            a = [z, z, z, z]
            cntv = z
            for g in range(G):
                mvec = msk_v[pl.ds(base + g * L, L)]
                nv = L
                if g == G - 1:
                    mvec = jnp.where(lane < TAIL, mvec, 0.0)
                    nv = TAIL
                cntv = cntv + mvec
                for i in range(nv):
                    t = g * L + i
                    m = mvec[i]
                    for kk in range(4):
                        a[kk] = a[kk] + buf[t, pl.ds(kk * L, L)] * m
            cnt = cntv[0]
            for i in range(1, L):
                cnt = cnt + cntv[i]
            denom = jnp.maximum(z + cnt, 1.0)
            for kk in range(4):
                out_v[pl.ds(j * D + kk * L, L)] = a[kk] / denom

        issue(0, 0, sem0)
        issue(1, 1, sem1)

        def step(s, carry):
            for half in range(2):
                j = 2 * s + half
                drain(half, sems[half])
                reduce_row(j, half)

                @pl.when(s < BPW // 2 - 1)
                def _():
                    issue(j + 2, half, sems[half])

            return carry

        lax.fori_loop(0, BPW // 2, step, 0)
        pltpu.sync_copy(out_v, out_hbm.at[pl.ds(wid * BPW * D, BPW * D)])

    return k


def kernel(token_ids, mask, table):
    B, T = token_ids.shape
    V, D = table.shape
    tok_flat = token_ids.astype(jnp.int32).reshape(-1)
    mask_flat = mask.astype(jnp.float32).reshape(-1)
    table_lin = _tc_transpose(V, D)(table.T).reshape(V, D)
    out_flat = _pooled_lookup(B, T, D)(tok_flat, mask_flat, table_lin)
    return out_flat.reshape(B, D)
